# baseline (speedup 1.0000x reference)
import functools

import jax
import jax.numpy as jnp
from jax import lax
from jax.experimental import pallas as pl
from jax.experimental.pallas import tpu as pltpu
from jax.experimental.pallas import tpu_sc as plsc

NB = 8192
SEM = 128
VQD = 128
ED = SEM + VQD
N_TOK = 4096
TM = 256
GRID = N_TOK // TM

SC_NC = 2
SC_NS = 16
SC_NW = SC_NC * SC_NS
BPW = N_TOK // SC_NW


def _vq_tc_kernel(z_ref, wkd_ref, wgan_ref,
                  e_ref, idx_ref, vq_ref, commit_ref, ent_ref, kdn_ref,
                  gann_ref,
                  acc_ref, avg_ref, esqkd_ref, esqgan_ref):
    i = pl.program_id(0)

    @pl.when(i == 0)
    def _init():
        wkd = wkd_ref[...]
        nkd = jnp.sqrt(jnp.sum(wkd * wkd, axis=1, keepdims=True))
        ekd = wkd / jnp.maximum(nkd, 1e-12)
        wgan = wgan_ref[...]
        ngan = jnp.sqrt(jnp.sum(wgan * wgan, axis=1, keepdims=True))
        egan = wgan / jnp.maximum(ngan, 1e-12)
        e_ref[:, :SEM] = ekd
        e_ref[:, SEM:] = egan
        ones = jnp.ones((1, SEM), jnp.float32)
        esqkd_ref[...] = lax.dot_general(
            ones, ekd * ekd, (((1,), (1,)), ((), ())),
            precision=lax.Precision.HIGHEST)
        esqgan_ref[...] = lax.dot_general(
            ones, egan * egan, (((1,), (1,)), ((), ())),
            precision=lax.Precision.HIGHEST)
        acc_ref[0] = 0.0
        acc_ref[1] = 0.0
        acc_ref[2] = 0.0
        acc_ref[3] = 0.0
        avg_ref[...] = jnp.zeros_like(avg_ref)

    ekd = e_ref[:, :SEM]
    egan = e_ref[:, SEM:]

    z = jnp.transpose(z_ref[0], (1, 0))
    zkd = z[:, :SEM]
    zgan = z[:, SEM:]
    nzkd = jnp.sqrt(jnp.sum(zkd * zkd, axis=1, keepdims=True))
    znkd = zkd / jnp.maximum(nzkd, 1e-12)
    nzgan = jnp.sqrt(jnp.sum(zgan * zgan, axis=1, keepdims=True))
    zngan = zgan / jnp.maximum(nzgan, 1e-12)
    zsqkd = jnp.sum(znkd * znkd, axis=1, keepdims=True)
    zsqgan = jnp.sum(zngan * zngan, axis=1, keepdims=True)

    mmkd2 = lax.dot_general(znkd * -2.0, ekd, (((1,), (1,)), ((), ())),
                            preferred_element_type=jnp.float32)
    mmgan2 = lax.dot_general(zngan * -2.0, egan, (((1,), (1,)), ((), ())),
                             preferred_element_type=jnp.float32)
    d_kd = (zsqkd + esqkd_ref[...]) + mmkd2
    d_gan = (zsqgan + esqgan_ref[...]) + mmgan2

    ones_nb = jnp.ones((1, NB), jnp.float32)

    rs_kd = lax.dot_general(d_kd * d_kd, ones_nb, (((1,), (1,)), ((), ())),
                            preferred_element_type=jnp.float32)
    rs_gan = lax.dot_general(d_gan * d_gan, ones_nb, (((1,), (1,)), ((), ())),
                             preferred_element_type=jnp.float32)
    acc_ref[1] += jnp.sum(rs_kd)
    acc_ref[2] += jnp.sum(rs_gan)

    d = d_kd + d_gan
    dmin = jnp.min(d, axis=1, keepdims=True)
    acc_ref[0] += jnp.sum(dmin)
    ii = lax.broadcasted_iota(jnp.int32, (TM, NB), 1)
    idx_ref[...] = jnp.min(jnp.where(d == dmin, ii, NB), axis=1,
                           keepdims=True)

    t = (dmin - d) * 100.0
    p_un = jnp.exp(t)
    zden = lax.dot_general(p_un, ones_nb, (((1,), (1,)), ((), ())),
                           preferred_element_type=jnp.float32)
    s_row = lax.dot_general(p_un * t, ones_nb, (((1,), (1,)), ((), ())),
                            preferred_element_type=jnp.float32)
    acc_ref[3] += jnp.sum(s_row / zden - jnp.log(zden))
    rz = 1.0 / zden
    avg_ref[...] += lax.dot_general(rz, p_un, (((0,), (0,)), ((), ())),
                                    preferred_element_type=jnp.float32)

    @pl.when(i == GRID - 1)
    def _fin():
        vq = acc_ref[0] / float(N_TOK * ED)
        vq_ref[...] = jnp.full((1, 1), vq, jnp.float32)
        commit_ref[...] = jnp.full((1, 1), 0.25 * vq, jnp.float32)
        kdn_ref[...] = jnp.full((1, 1), acc_ref[1] / float(N_TOK),
                                jnp.float32)
        gann_ref[...] = jnp.full((1, 1), acc_ref[2] / float(N_TOK),
                                 jnp.float32)
        ap = avg_ref[...] / float(N_TOK)
        avg_entropy = 0.0 - jnp.sum(ap * jnp.log(ap + 1e-05))
        sample_entropy = 0.0 - acc_ref[3] / float(N_TOK)
        ent_ref[...] = jnp.full((1, 1), 0.1 * (sample_entropy - avg_entropy),
                                jnp.float32)


def _tc_call(z_flat, w_vqkd, w_vqgan, interpret=False):
    f32 = jnp.float32
    return pl.pallas_call(
        _vq_tc_kernel,
        grid=(GRID,),
        in_specs=[
            pl.BlockSpec((1, ED, TM), lambda i: (i // 4, 0, i % 4)),
            pl.BlockSpec((NB, SEM), lambda i: (0, 0)),
            pl.BlockSpec((NB, VQD), lambda i: (0, 0)),
        ],
        out_specs=[
            pl.BlockSpec((NB, ED), lambda i: (0, 0)),
            pl.BlockSpec((TM, 1), lambda i: (i, 0)),
            pl.BlockSpec((1, 1), lambda i: (0, 0)),
            pl.BlockSpec((1, 1), lambda i: (0, 0)),
            pl.BlockSpec((1, 1), lambda i: (0, 0)),
            pl.BlockSpec((1, 1), lambda i: (0, 0)),
            pl.BlockSpec((1, 1), lambda i: (0, 0)),
        ],
        out_shape=[
            jax.ShapeDtypeStruct((NB, ED), f32),
            jax.ShapeDtypeStruct((N_TOK, 1), jnp.int32),
            jax.ShapeDtypeStruct((1, 1), f32),
            jax.ShapeDtypeStruct((1, 1), f32),
            jax.ShapeDtypeStruct((1, 1), f32),
            jax.ShapeDtypeStruct((1, 1), f32),
            jax.ShapeDtypeStruct((1, 1), f32),
        ],
        scratch_shapes=[
            pltpu.SMEM((8,), f32),
            pltpu.VMEM((1, NB), f32),
            pltpu.VMEM((1, NB), f32),
            pltpu.VMEM((1, NB), f32),
        ],
        interpret=interpret,
    )(z_flat, w_vqkd, w_vqgan)


@functools.cache
def _sc_gather_fn():
    mesh = plsc.VectorSubcoreMesh(core_axis_name="c", subcore_axis_name="s")

    @functools.partial(
        pl.kernel,
        mesh=mesh,
        out_type=jax.ShapeDtypeStruct((N_TOK, ED), jnp.float32),
        scratch_types=[
            pltpu.VMEM((BPW,), jnp.int32),
            pltpu.VMEM((BPW, ED), jnp.float32),
            pltpu.SemaphoreType.DMA,
        ],
    )
    def gather(table_hbm, idx_hbm, out_hbm, idx_v, rows_v, sem):
        wid = lax.axis_index("s") * SC_NC + lax.axis_index("c")
        base = wid * BPW
        pltpu.sync_copy(idx_hbm.at[pl.ds(base, BPW)], idx_v)
        pltpu.async_copy(table_hbm.at[idx_v], rows_v, sem).wait()
        pltpu.sync_copy(rows_v, out_hbm.at[pl.ds(base, BPW)])

    return gather


def kernel(z, w_vqkd, w_vqgan):
    z3 = z.reshape(4, ED, 1024)
    e_all, idx2, vq, commit, ent, kdn, gann = _tc_call(z3, w_vqkd,
                                                       w_vqgan)
    idx = idx2.reshape(N_TOK)
    z_qf = _sc_gather_fn()(e_all, idx)
    z_q_out = jnp.transpose(z_qf.reshape(4, 32, 32, ED), (0, 3, 1, 2))
    return (z_q_out, vq.reshape(()), commit.reshape(()), ent.reshape(()),
            kdn.reshape(()), gann.reshape(()), idx)


# speedup vs baseline: 1.1029x; 1.1029x over previous
import functools

import jax
import jax.numpy as jnp
from jax import lax
from jax.experimental import pallas as pl
from jax.experimental.pallas import tpu as pltpu
from jax.experimental.pallas import tpu_sc as plsc

NB = 8192
SEM = 128
VQD = 128
ED = SEM + VQD
N_TOK = 4096
TM = 256
GRID = N_TOK // TM

SC_NC = 2
SC_NS = 16
SC_NW = SC_NC * SC_NS
BPW = N_TOK // SC_NW


def _vq_tc_kernel(z_ref, wkd_ref, wgan_ref,
                  e_ref, idx_ref, vq_ref, commit_ref, ent_ref, kdn_ref,
                  gann_ref,
                  acc_ref, avg_ref, esqkd_ref, esqgan_ref):
    i = pl.program_id(0)

    @pl.when(i == 0)
    def _init():
        wkd = wkd_ref[...]
        nkd = jnp.sqrt(jnp.sum(wkd * wkd, axis=1, keepdims=True))
        ekd = wkd / jnp.maximum(nkd, 1e-12)
        wgan = wgan_ref[...]
        ngan = jnp.sqrt(jnp.sum(wgan * wgan, axis=1, keepdims=True))
        egan = wgan / jnp.maximum(ngan, 1e-12)
        e_ref[:, :SEM] = ekd
        e_ref[:, SEM:] = egan
        ones = jnp.ones((1, SEM), jnp.float32)
        esqkd_ref[...] = lax.dot_general(
            ones, ekd * ekd, (((1,), (1,)), ((), ())),
            precision=lax.Precision.HIGHEST)
        esqgan_ref[...] = lax.dot_general(
            ones, egan * egan, (((1,), (1,)), ((), ())),
            precision=lax.Precision.HIGHEST)
        acc_ref[0] = 0.0
        acc_ref[1] = 0.0
        acc_ref[2] = 0.0
        acc_ref[3] = 0.0
        avg_ref[...] = jnp.zeros_like(avg_ref)

    ekd = e_ref[:, :SEM]
    egan = e_ref[:, SEM:]

    z = z_ref[...]
    zkd = z[:, :SEM]
    zgan = z[:, SEM:]
    nzkd = jnp.sqrt(jnp.sum(zkd * zkd, axis=1, keepdims=True))
    znkd = zkd / jnp.maximum(nzkd, 1e-12)
    nzgan = jnp.sqrt(jnp.sum(zgan * zgan, axis=1, keepdims=True))
    zngan = zgan / jnp.maximum(nzgan, 1e-12)
    zsqkd = jnp.sum(znkd * znkd, axis=1, keepdims=True)
    zsqgan = jnp.sum(zngan * zngan, axis=1, keepdims=True)

    mmkd2 = lax.dot_general(znkd * -2.0, ekd, (((1,), (1,)), ((), ())),
                            preferred_element_type=jnp.float32)
    mmgan2 = lax.dot_general(zngan * -2.0, egan, (((1,), (1,)), ((), ())),
                             preferred_element_type=jnp.float32)
    d_kd = (zsqkd + esqkd_ref[...]) + mmkd2
    d_gan = (zsqgan + esqgan_ref[...]) + mmgan2

    ones_nb = jnp.ones((1, NB), jnp.float32)

    rs_kd = lax.dot_general(d_kd * d_kd, ones_nb, (((1,), (1,)), ((), ())),
                            preferred_element_type=jnp.float32)
    rs_gan = lax.dot_general(d_gan * d_gan, ones_nb, (((1,), (1,)), ((), ())),
                             preferred_element_type=jnp.float32)
    acc_ref[1] += jnp.sum(rs_kd)
    acc_ref[2] += jnp.sum(rs_gan)

    d = d_kd + d_gan
    dmin = jnp.min(d, axis=1, keepdims=True)
    acc_ref[0] += jnp.sum(dmin)
    ii = lax.broadcasted_iota(jnp.int32, (TM, NB), 1)
    idx_ref[...] = jnp.min(jnp.where(d == dmin, ii, NB), axis=1,
                           keepdims=True)

    t2 = (dmin - d) * 144.26950408889634
    p_un = jnp.exp2(t2)
    zden = lax.dot_general(p_un, ones_nb, (((1,), (1,)), ((), ())),
                           preferred_element_type=jnp.float32)
    s2_row = lax.dot_general(p_un * t2, ones_nb, (((1,), (1,)), ((), ())),
                             preferred_element_type=jnp.float32)
    s_row = s2_row * 0.6931471805599453
    acc_ref[3] += jnp.sum(s_row / zden - jnp.log(zden))
    rz = 1.0 / zden
    avg_ref[...] += lax.dot_general(rz, p_un, (((0,), (0,)), ((), ())),
                                    preferred_element_type=jnp.float32)

    @pl.when(i == GRID - 1)
    def _fin():
        vq = acc_ref[0] / float(N_TOK * ED)
        vq_ref[...] = jnp.full((1, 1), vq, jnp.float32)
        commit_ref[...] = jnp.full((1, 1), 0.25 * vq, jnp.float32)
        kdn_ref[...] = jnp.full((1, 1), acc_ref[1] / float(N_TOK),
                                jnp.float32)
        gann_ref[...] = jnp.full((1, 1), acc_ref[2] / float(N_TOK),
                                 jnp.float32)
        ap = avg_ref[...] / float(N_TOK)
        avg_entropy = 0.0 - jnp.sum(ap * jnp.log(ap + 1e-05))
        sample_entropy = 0.0 - acc_ref[3] / float(N_TOK)
        ent_ref[...] = jnp.full((1, 1), 0.1 * (sample_entropy - avg_entropy),
                                jnp.float32)


def _tc_call(z_flat, w_vqkd, w_vqgan, interpret=False):
    f32 = jnp.float32
    return pl.pallas_call(
        _vq_tc_kernel,
        grid=(GRID,),
        in_specs=[
            pl.BlockSpec((TM, ED), lambda i: (i, 0)),
            pl.BlockSpec((NB, SEM), lambda i: (0, 0)),
            pl.BlockSpec((NB, VQD), lambda i: (0, 0)),
        ],
        out_specs=[
            pl.BlockSpec((NB, ED), lambda i: (0, 0)),
            pl.BlockSpec((TM, 1), lambda i: (i, 0)),
            pl.BlockSpec((1, 1), lambda i: (0, 0)),
            pl.BlockSpec((1, 1), lambda i: (0, 0)),
            pl.BlockSpec((1, 1), lambda i: (0, 0)),
            pl.BlockSpec((1, 1), lambda i: (0, 0)),
            pl.BlockSpec((1, 1), lambda i: (0, 0)),
        ],
        out_shape=[
            jax.ShapeDtypeStruct((NB, ED), f32),
            jax.ShapeDtypeStruct((N_TOK, 1), jnp.int32),
            jax.ShapeDtypeStruct((1, 1), f32),
            jax.ShapeDtypeStruct((1, 1), f32),
            jax.ShapeDtypeStruct((1, 1), f32),
            jax.ShapeDtypeStruct((1, 1), f32),
            jax.ShapeDtypeStruct((1, 1), f32),
        ],
        scratch_shapes=[
            pltpu.SMEM((8,), f32),
            pltpu.VMEM((1, NB), f32),
            pltpu.VMEM((1, NB), f32),
            pltpu.VMEM((1, NB), f32),
        ],
        interpret=interpret,
    )(z_flat, w_vqkd, w_vqgan)


@functools.cache
def _sc_gather_fn():
    mesh = plsc.VectorSubcoreMesh(core_axis_name="c", subcore_axis_name="s")

    @functools.partial(
        pl.kernel,
        mesh=mesh,
        out_type=jax.ShapeDtypeStruct((N_TOK, ED), jnp.float32),
        scratch_types=[
            pltpu.VMEM((BPW,), jnp.int32),
            pltpu.VMEM((BPW, ED), jnp.float32),
            pltpu.SemaphoreType.DMA,
        ],
    )
    def gather(table_hbm, idx_hbm, out_hbm, idx_v, rows_v, sem):
        wid = lax.axis_index("s") * SC_NC + lax.axis_index("c")
        base = wid * BPW
        pltpu.sync_copy(idx_hbm.at[pl.ds(base, BPW)], idx_v)
        pltpu.async_copy(table_hbm.at[idx_v], rows_v, sem).wait()
        pltpu.sync_copy(rows_v, out_hbm.at[pl.ds(base, BPW)])

    return gather


def kernel(z, w_vqkd, w_vqgan):
    zp = jnp.transpose(z, (0, 2, 3, 1))
    z_flat = zp.reshape(N_TOK, ED)
    e_all, idx2, vq, commit, ent, kdn, gann = _tc_call(z_flat, w_vqkd,
                                                       w_vqgan)
    idx = idx2.reshape(N_TOK)
    z_qf = _sc_gather_fn()(e_all, idx)
    z_q_out = jnp.transpose(z_qf.reshape(4, 32, 32, ED), (0, 3, 1, 2))
    return (z_q_out, vq.reshape(()), commit.reshape(()), ent.reshape(()),
            kdn.reshape(()), gann.reshape(()), idx)


# speedup vs baseline: 1.1400x; 1.0336x over previous
import functools

import jax
import jax.numpy as jnp
from jax import lax
from jax.experimental import pallas as pl
from jax.experimental.pallas import tpu as pltpu
from jax.experimental.pallas import tpu_sc as plsc

NB = 8192
SEM = 128
VQD = 128
ED = SEM + VQD
N_TOK = 4096
TM = 256
GRID = N_TOK // TM

SC_NC = 2
SC_NS = 16
SC_NW = SC_NC * SC_NS
BPW = N_TOK // SC_NW


def _vq_tc_kernel(z_ref, wkd_ref, wgan_ref,
                  e_ref, idx_ref, vq_ref, commit_ref, ent_ref, kdn_ref,
                  gann_ref,
                  acc_ref, avg_ref, esqkd_ref, esqgan_ref,
                  wkd2_ref, wgan2_ref, gkd_ref, ggan_ref):
    i = pl.program_id(0)

    @pl.when(i == 0)
    def _init():
        wkd = wkd_ref[...]
        nkd = jnp.sqrt(jnp.sum(wkd * wkd, axis=1, keepdims=True))
        ekd = wkd / jnp.maximum(nkd, 1e-12)
        wgan = wgan_ref[...]
        ngan = jnp.sqrt(jnp.sum(wgan * wgan, axis=1, keepdims=True))
        egan = wgan / jnp.maximum(ngan, 1e-12)
        e_ref[:, :SEM] = ekd
        e_ref[:, SEM:] = egan
        ones = jnp.ones((1, SEM), jnp.float32)
        esqkd_ref[...] = lax.dot_general(
            ones, ekd * ekd, (((1,), (1,)), ((), ())),
            precision=lax.Precision.HIGHEST)
        esqgan_ref[...] = lax.dot_general(
            ones, egan * egan, (((1,), (1,)), ((), ())),
            precision=lax.Precision.HIGHEST)
        onesnb = jnp.ones((1, NB), jnp.float32)
        w1kd = lax.dot_general(onesnb, ekd, (((1,), (0,)), ((), ())),
                               preferred_element_type=jnp.float32) * -2.0
        w2kd = lax.dot_general(esqkd_ref[...], ekd, (((1,), (0,)), ((), ())),
                               preferred_element_type=jnp.float32) * -2.0
        wkd2_ref[...] = jnp.concatenate([w1kd, w2kd], axis=0)
        w1gan = lax.dot_general(onesnb, egan, (((1,), (0,)), ((), ())),
                                preferred_element_type=jnp.float32) * -2.0
        w2gan = lax.dot_general(esqgan_ref[...], egan,
                                (((1,), (0,)), ((), ())),
                                preferred_element_type=jnp.float32) * -2.0
        wgan2_ref[...] = jnp.concatenate([w1gan, w2gan], axis=0)
        gkd_ref[...] = lax.dot_general(ekd, ekd, (((0,), (0,)), ((), ())),
                                       preferred_element_type=jnp.float32)
        ggan_ref[...] = lax.dot_general(egan, egan, (((0,), (0,)), ((), ())),
                                        preferred_element_type=jnp.float32)
        acc_ref[0] = 0.0
        acc_ref[1] = 0.0
        acc_ref[2] = 0.0
        acc_ref[3] = 0.0
        acc_ref[4] = jnp.sum(esqkd_ref[...])
        acc_ref[5] = jnp.sum(esqkd_ref[...] * esqkd_ref[...])
        acc_ref[6] = jnp.sum(esqgan_ref[...])
        acc_ref[7] = jnp.sum(esqgan_ref[...] * esqgan_ref[...])
        avg_ref[...] = jnp.zeros_like(avg_ref)

    ekd = e_ref[:, :SEM]
    egan = e_ref[:, SEM:]

    z = z_ref[...]
    zkd = z[:, :SEM]
    zgan = z[:, SEM:]
    nzkd = jnp.sqrt(jnp.sum(zkd * zkd, axis=1, keepdims=True))
    znkd = zkd / jnp.maximum(nzkd, 1e-12)
    nzgan = jnp.sqrt(jnp.sum(zgan * zgan, axis=1, keepdims=True))
    zngan = zgan / jnp.maximum(nzgan, 1e-12)
    zsqkd = jnp.sum(znkd * znkd, axis=1, keepdims=True)
    zsqgan = jnp.sum(zngan * zngan, axis=1, keepdims=True)

    mmkd2 = lax.dot_general(znkd * -2.0, ekd, (((1,), (1,)), ((), ())),
                            preferred_element_type=jnp.float32)
    mmgan2 = lax.dot_general(zngan * -2.0, egan, (((1,), (1,)), ((), ())),
                             preferred_element_type=jnp.float32)
    d_kd = (zsqkd + esqkd_ref[...]) + mmkd2
    d_gan = (zsqgan + esqgan_ref[...]) + mmgan2

    ones_nb = jnp.ones((1, NB), jnp.float32)

    cwkd = lax.dot_general(znkd, wkd2_ref[...], (((1,), (1,)), ((), ())),
                           preferred_element_type=jnp.float32)
    gzkd = lax.dot_general(znkd, gkd_ref[...], (((1,), (0,)), ((), ())),
                           preferred_element_type=jnp.float32)
    qkd = jnp.sum(gzkd * znkd, axis=1, keepdims=True)
    rs_kd = (float(NB) * zsqkd * zsqkd
             + 2.0 * acc_ref[4] * zsqkd + acc_ref[5]
             + 2.0 * (zsqkd * cwkd[:, 0:1] + cwkd[:, 1:2]) + 4.0 * qkd)
    cwgan = lax.dot_general(zngan, wgan2_ref[...], (((1,), (1,)), ((), ())),
                            preferred_element_type=jnp.float32)
    gzgan = lax.dot_general(zngan, ggan_ref[...], (((1,), (0,)), ((), ())),
                            preferred_element_type=jnp.float32)
    qgan = jnp.sum(gzgan * zngan, axis=1, keepdims=True)
    rs_gan = (float(NB) * zsqgan * zsqgan
              + 2.0 * acc_ref[6] * zsqgan + acc_ref[7]
              + 2.0 * (zsqgan * cwgan[:, 0:1] + cwgan[:, 1:2]) + 4.0 * qgan)
    acc_ref[1] += jnp.sum(rs_kd)
    acc_ref[2] += jnp.sum(rs_gan)

    d = d_kd + d_gan
    dmin = jnp.min(d, axis=1, keepdims=True)
    acc_ref[0] += jnp.sum(dmin)
    ii = lax.broadcasted_iota(jnp.int32, (TM, NB), 1)
    idx_ref[...] = jnp.min(jnp.where(d == dmin, ii, NB), axis=1,
                           keepdims=True)

    t2 = (dmin - d) * 144.26950408889634
    p_un = jnp.exp2(t2)
    zden = lax.dot_general(p_un, ones_nb, (((1,), (1,)), ((), ())),
                           preferred_element_type=jnp.float32)
    s2_row = lax.dot_general(p_un * t2, ones_nb, (((1,), (1,)), ((), ())),
                             preferred_element_type=jnp.float32)
    s_row = s2_row * 0.6931471805599453
    acc_ref[3] += jnp.sum(s_row / zden - jnp.log(zden))
    rz = 1.0 / zden
    avg_ref[...] += lax.dot_general(rz, p_un, (((0,), (0,)), ((), ())),
                                    preferred_element_type=jnp.float32)

    @pl.when(i == GRID - 1)
    def _fin():
        vq = acc_ref[0] / float(N_TOK * ED)
        vq_ref[...] = jnp.full((1, 1), vq, jnp.float32)
        commit_ref[...] = jnp.full((1, 1), 0.25 * vq, jnp.float32)
        kdn_ref[...] = jnp.full((1, 1), acc_ref[1] / float(N_TOK),
                                jnp.float32)
        gann_ref[...] = jnp.full((1, 1), acc_ref[2] / float(N_TOK),
                                 jnp.float32)
        ap = avg_ref[...] / float(N_TOK)
        avg_entropy = 0.0 - jnp.sum(ap * jnp.log(ap + 1e-05))
        sample_entropy = 0.0 - acc_ref[3] / float(N_TOK)
        ent_ref[...] = jnp.full((1, 1), 0.1 * (sample_entropy - avg_entropy),
                                jnp.float32)


def _tc_call(z_flat, w_vqkd, w_vqgan, interpret=False):
    f32 = jnp.float32
    return pl.pallas_call(
        _vq_tc_kernel,
        grid=(GRID,),
        in_specs=[
            pl.BlockSpec((TM, ED), lambda i: (i, 0)),
            pl.BlockSpec((NB, SEM), lambda i: (0, 0)),
            pl.BlockSpec((NB, VQD), lambda i: (0, 0)),
        ],
        out_specs=[
            pl.BlockSpec((NB, ED), lambda i: (0, 0)),
            pl.BlockSpec((TM, 1), lambda i: (i, 0)),
            pl.BlockSpec((1, 1), lambda i: (0, 0)),
            pl.BlockSpec((1, 1), lambda i: (0, 0)),
            pl.BlockSpec((1, 1), lambda i: (0, 0)),
            pl.BlockSpec((1, 1), lambda i: (0, 0)),
            pl.BlockSpec((1, 1), lambda i: (0, 0)),
        ],
        out_shape=[
            jax.ShapeDtypeStruct((NB, ED), f32),
            jax.ShapeDtypeStruct((N_TOK, 1), jnp.int32),
            jax.ShapeDtypeStruct((1, 1), f32),
            jax.ShapeDtypeStruct((1, 1), f32),
            jax.ShapeDtypeStruct((1, 1), f32),
            jax.ShapeDtypeStruct((1, 1), f32),
            jax.ShapeDtypeStruct((1, 1), f32),
        ],
        scratch_shapes=[
            pltpu.SMEM((8,), f32),
            pltpu.VMEM((1, NB), f32),
            pltpu.VMEM((1, NB), f32),
            pltpu.VMEM((1, NB), f32),
            pltpu.VMEM((2, SEM), f32),
            pltpu.VMEM((2, VQD), f32),
            pltpu.VMEM((SEM, SEM), f32),
            pltpu.VMEM((VQD, VQD), f32),
        ],
        interpret=interpret,
    )(z_flat, w_vqkd, w_vqgan)


@functools.cache
def _sc_gather_fn():
    mesh = plsc.VectorSubcoreMesh(core_axis_name="c", subcore_axis_name="s")

    @functools.partial(
        pl.kernel,
        mesh=mesh,
        out_type=jax.ShapeDtypeStruct((N_TOK, ED), jnp.float32),
        scratch_types=[
            pltpu.VMEM((BPW,), jnp.int32),
            pltpu.VMEM((BPW, ED), jnp.float32),
            pltpu.SemaphoreType.DMA,
        ],
    )
    def gather(table_hbm, idx_hbm, out_hbm, idx_v, rows_v, sem):
        wid = lax.axis_index("s") * SC_NC + lax.axis_index("c")
        base = wid * BPW
        pltpu.sync_copy(idx_hbm.at[pl.ds(base, BPW)], idx_v)
        pltpu.async_copy(table_hbm.at[idx_v], rows_v, sem).wait()
        pltpu.sync_copy(rows_v, out_hbm.at[pl.ds(base, BPW)])

    return gather


def kernel(z, w_vqkd, w_vqgan):
    zp = jnp.transpose(z, (0, 2, 3, 1))
    z_flat = zp.reshape(N_TOK, ED)
    e_all, idx2, vq, commit, ent, kdn, gann = _tc_call(z_flat, w_vqkd,
                                                       w_vqgan)
    idx = idx2.reshape(N_TOK)
    z_qf = _sc_gather_fn()(e_all, idx)
    z_q_out = jnp.transpose(z_qf.reshape(4, 32, 32, ED), (0, 3, 1, 2))
    return (z_q_out, vq.reshape(()), commit.reshape(()), ent.reshape(()),
            kdn.reshape(()), gann.reshape(()), idx)


# speedup vs baseline: 1.1404x; 1.0004x over previous
import functools

import jax
import jax.numpy as jnp
from jax import lax
from jax.experimental import pallas as pl
from jax.experimental.pallas import tpu as pltpu
from jax.experimental.pallas import tpu_sc as plsc

NB = 8192
SEM = 128
VQD = 128
ED = SEM + VQD
N_TOK = 4096
TM = 256
GRID = N_TOK // TM

SC_NC = 2
SC_NS = 16
SC_NW = SC_NC * SC_NS
BPW = N_TOK // SC_NW


def _vq_tc_kernel(z_ref, wkd_ref, wgan_ref,
                  e_ref, idx_ref, vq_ref, commit_ref, ent_ref, kdn_ref,
                  gann_ref,
                  acc_ref, avg_ref, esqkd_ref, esqgan_ref,
                  wkd2_ref, wgan2_ref, gkd_ref, ggan_ref):
    i = pl.program_id(0)

    @pl.when(i == 0)
    def _init():
        wkd = wkd_ref[...]
        nkd = jnp.sqrt(jnp.sum(wkd * wkd, axis=1, keepdims=True))
        ekd = wkd / jnp.maximum(nkd, 1e-12)
        wgan = wgan_ref[...]
        ngan = jnp.sqrt(jnp.sum(wgan * wgan, axis=1, keepdims=True))
        egan = wgan / jnp.maximum(ngan, 1e-12)
        e_ref[:, :SEM] = ekd
        e_ref[:, SEM:] = egan
        ones = jnp.ones((1, SEM), jnp.float32)
        esqkd_ref[...] = lax.dot_general(
            ones, ekd * ekd, (((1,), (1,)), ((), ())),
            precision=lax.Precision.HIGHEST)
        esqgan_ref[...] = lax.dot_general(
            ones, egan * egan, (((1,), (1,)), ((), ())),
            precision=lax.Precision.HIGHEST)
        onesnb = jnp.ones((1, NB), jnp.float32)
        w1kd = lax.dot_general(onesnb, ekd, (((1,), (0,)), ((), ())),
                               preferred_element_type=jnp.float32) * -2.0
        w2kd = lax.dot_general(esqkd_ref[...], ekd, (((1,), (0,)), ((), ())),
                               preferred_element_type=jnp.float32) * -2.0
        wkd2_ref[...] = jnp.concatenate([w1kd, w2kd], axis=0)
        w1gan = lax.dot_general(onesnb, egan, (((1,), (0,)), ((), ())),
                                preferred_element_type=jnp.float32) * -2.0
        w2gan = lax.dot_general(esqgan_ref[...], egan,
                                (((1,), (0,)), ((), ())),
                                preferred_element_type=jnp.float32) * -2.0
        wgan2_ref[...] = jnp.concatenate([w1gan, w2gan], axis=0)
        gkd_ref[...] = lax.dot_general(ekd, ekd, (((0,), (0,)), ((), ())),
                                       preferred_element_type=jnp.float32)
        ggan_ref[...] = lax.dot_general(egan, egan, (((0,), (0,)), ((), ())),
                                        preferred_element_type=jnp.float32)
        acc_ref[0] = 0.0
        acc_ref[1] = 0.0
        acc_ref[2] = 0.0
        acc_ref[3] = 0.0
        acc_ref[4] = jnp.sum(esqkd_ref[...])
        acc_ref[5] = jnp.sum(esqkd_ref[...] * esqkd_ref[...])
        acc_ref[6] = jnp.sum(esqgan_ref[...])
        acc_ref[7] = jnp.sum(esqgan_ref[...] * esqgan_ref[...])
        avg_ref[...] = jnp.zeros_like(avg_ref)

    ekd = e_ref[:, :SEM]
    egan = e_ref[:, SEM:]

    z = z_ref[...]
    zkd = z[:, :SEM]
    zgan = z[:, SEM:]
    nzkd = jnp.sqrt(jnp.sum(zkd * zkd, axis=1, keepdims=True))
    znkd = zkd / jnp.maximum(nzkd, 1e-12)
    nzgan = jnp.sqrt(jnp.sum(zgan * zgan, axis=1, keepdims=True))
    zngan = zgan / jnp.maximum(nzgan, 1e-12)
    zsqkd = jnp.sum(znkd * znkd, axis=1, keepdims=True)
    zsqgan = jnp.sum(zngan * zngan, axis=1, keepdims=True)

    mmkd2 = lax.dot_general(znkd * -2.0, ekd, (((1,), (1,)), ((), ())),
                            preferred_element_type=jnp.float32)
    mmgan2 = lax.dot_general(zngan * -2.0, egan, (((1,), (1,)), ((), ())),
                             preferred_element_type=jnp.float32)
    d_kd = (zsqkd + esqkd_ref[...]) + mmkd2
    d_gan = (zsqgan + esqgan_ref[...]) + mmgan2

    ones_nb = jnp.ones((1, NB), jnp.float32)

    cwkd = lax.dot_general(znkd, wkd2_ref[...], (((1,), (1,)), ((), ())),
                           preferred_element_type=jnp.float32)
    gzkd = lax.dot_general(znkd, gkd_ref[...], (((1,), (0,)), ((), ())),
                           preferred_element_type=jnp.float32)
    qkd = jnp.sum(gzkd * znkd, axis=1, keepdims=True)
    rs_kd = (float(NB) * zsqkd * zsqkd
             + 2.0 * acc_ref[4] * zsqkd + acc_ref[5]
             + 2.0 * (zsqkd * cwkd[:, 0:1] + cwkd[:, 1:2]) + 4.0 * qkd)
    cwgan = lax.dot_general(zngan, wgan2_ref[...], (((1,), (1,)), ((), ())),
                            preferred_element_type=jnp.float32)
    gzgan = lax.dot_general(zngan, ggan_ref[...], (((1,), (0,)), ((), ())),
                            preferred_element_type=jnp.float32)
    qgan = jnp.sum(gzgan * zngan, axis=1, keepdims=True)
    rs_gan = (float(NB) * zsqgan * zsqgan
              + 2.0 * acc_ref[6] * zsqgan + acc_ref[7]
              + 2.0 * (zsqgan * cwgan[:, 0:1] + cwgan[:, 1:2]) + 4.0 * qgan)
    acc_ref[1] += jnp.sum(rs_kd)
    acc_ref[2] += jnp.sum(rs_gan)

    d = d_kd + d_gan
    dmin = jnp.min(d, axis=1, keepdims=True)
    acc_ref[0] += jnp.sum(dmin)
    ii = lax.broadcasted_iota(jnp.int32, (TM, NB), 1)
    idx_ref[...] = jnp.min(jnp.where(d == dmin, ii, NB), axis=1,
                           keepdims=True)

    t2 = (dmin - d) * 144.26950408889634
    p_un = jnp.exp2(t2)
    zden = lax.dot_general(p_un, ones_nb, (((1,), (1,)), ((), ())),
                           preferred_element_type=jnp.float32)
    s2_row = lax.dot_general(p_un * t2, ones_nb, (((1,), (1,)), ((), ())),
                             preferred_element_type=jnp.float32)
    s_row = s2_row * 0.6931471805599453
    acc_ref[3] += jnp.sum(s_row / zden - jnp.log(zden))
    rz = 1.0 / zden
    avg_ref[...] += lax.dot_general(rz, p_un, (((0,), (0,)), ((), ())),
                                    preferred_element_type=jnp.float32)

    @pl.when(i == GRID - 1)
    def _fin():
        vq = acc_ref[0] / float(N_TOK * ED)
        vq_ref[...] = jnp.full((1, 1), vq, jnp.float32)
        commit_ref[...] = jnp.full((1, 1), 0.25 * vq, jnp.float32)
        kdn_ref[...] = jnp.full((1, 1), acc_ref[1] / float(N_TOK),
                                jnp.float32)
        gann_ref[...] = jnp.full((1, 1), acc_ref[2] / float(N_TOK),
                                 jnp.float32)
        ap = avg_ref[...] / float(N_TOK)
        avg_entropy = 0.0 - jnp.sum(ap * jnp.log(ap + 1e-05))
        sample_entropy = 0.0 - acc_ref[3] / float(N_TOK)
        ent_ref[...] = jnp.full((1, 1), 0.1 * (sample_entropy - avg_entropy),
                                jnp.float32)


def _tc_call(z_flat, w_vqkd, w_vqgan):
    f32 = jnp.float32
    return pl.pallas_call(
        _vq_tc_kernel,
        grid=(GRID,),
        in_specs=[
            pl.BlockSpec((TM, ED), lambda i: (i, 0)),
            pl.BlockSpec((NB, SEM), lambda i: (0, 0)),
            pl.BlockSpec((NB, VQD), lambda i: (0, 0)),
        ],
        out_specs=[
            pl.BlockSpec((NB, ED), lambda i: (0, 0)),
            pl.BlockSpec((TM, 1), lambda i: (i, 0)),
            pl.BlockSpec((1, 1), lambda i: (0, 0)),
            pl.BlockSpec((1, 1), lambda i: (0, 0)),
            pl.BlockSpec((1, 1), lambda i: (0, 0)),
            pl.BlockSpec((1, 1), lambda i: (0, 0)),
            pl.BlockSpec((1, 1), lambda i: (0, 0)),
        ],
        out_shape=[
            jax.ShapeDtypeStruct((NB, ED), f32),
            jax.ShapeDtypeStruct((N_TOK, 1), jnp.int32),
            jax.ShapeDtypeStruct((1, 1), f32),
            jax.ShapeDtypeStruct((1, 1), f32),
            jax.ShapeDtypeStruct((1, 1), f32),
            jax.ShapeDtypeStruct((1, 1), f32),
            jax.ShapeDtypeStruct((1, 1), f32),
        ],
        scratch_shapes=[
            pltpu.SMEM((8,), f32),
            pltpu.VMEM((1, NB), f32),
            pltpu.VMEM((1, NB), f32),
            pltpu.VMEM((1, NB), f32),
            pltpu.VMEM((2, SEM), f32),
            pltpu.VMEM((2, VQD), f32),
            pltpu.VMEM((SEM, SEM), f32),
            pltpu.VMEM((VQD, VQD), f32),
        ],
    )(z_flat, w_vqkd, w_vqgan)


@functools.cache
def _sc_gather_fn():
    mesh = plsc.VectorSubcoreMesh(core_axis_name="c", subcore_axis_name="s")

    @functools.partial(
        pl.kernel,
        mesh=mesh,
        out_type=jax.ShapeDtypeStruct((N_TOK, ED), jnp.float32),
        scratch_types=[
            pltpu.VMEM((BPW,), jnp.int32),
            pltpu.VMEM((BPW, ED), jnp.float32),
            pltpu.SemaphoreType.DMA,
        ],
    )
    def gather(table_hbm, idx_hbm, out_hbm, idx_v, rows_v, sem):
        wid = lax.axis_index("s") * SC_NC + lax.axis_index("c")
        base = wid * BPW
        pltpu.sync_copy(idx_hbm.at[pl.ds(base, BPW)], idx_v)
        pltpu.async_copy(table_hbm.at[idx_v], rows_v, sem).wait()
        pltpu.sync_copy(rows_v, out_hbm.at[pl.ds(base, BPW)])

    return gather


def kernel(z, w_vqkd, w_vqgan):
    zp = jnp.transpose(z, (0, 2, 3, 1))
    z_flat = zp.reshape(N_TOK, ED)
    e_all, idx2, vq, commit, ent, kdn, gann = _tc_call(z_flat, w_vqkd,
                                                       w_vqgan)
    idx = idx2.reshape(N_TOK)
    z_qf = _sc_gather_fn()(e_all, idx)
    z_q_out = jnp.transpose(z_qf.reshape(4, 32, 32, ED), (0, 3, 1, 2))
    return (z_q_out, vq.reshape(()), commit.reshape(()), ent.reshape(()),
            kdn.reshape(()), gann.reshape(()), idx)
